# baseline (speedup 1.0000x reference)
import functools

import jax
import jax.numpy as jnp
from jax import lax
from jax.experimental import pallas as pl
from jax.experimental.pallas import tpu as pltpu
from jax.experimental.pallas import tpu_sc as plsc

N_NODES = 10000
N_EDGES = 320000
NC = 2
NS = 16
NW = NC * NS
CHUNK = 128
N_ROWS = N_EDGES // CHUNK
ROWS_PER_W = N_ROWS // NW
TAIL_ROW0 = NW * ROWS_PER_W
TAIL_WID0 = NW - (N_ROWS - TAIL_ROW0)
NBUF = 6
DEG_W = 16
TILE_ROWS = 624
TAIL_BASE = NS * TILE_ROWS
TAIL = N_NODES - TAIL_BASE

_mesh = plsc.VectorSubcoreMesh(core_axis_name="c", subcore_axis_name="s")
_sc_params = pltpu.CompilerParams(use_tc_tiling_on_sc=False)


def _copy_tile_slice(s, src, dst):
    pltpu.sync_copy(src.at[pl.ds(s * TILE_ROWS, TILE_ROWS)],
                    dst.at[pl.ds(s * TILE_ROWS, TILE_ROWS)])

    @pl.when(s == NS - 1)
    def _():
        pltpu.sync_copy(src.at[pl.ds(TAIL_BASE, TAIL)],
                        dst.at[pl.ds(TAIL_BASE, TAIL)])


EDGES_PER_W = ROWS_PER_W * CHUNK


def _load_index_flat(wid, ei_hbm, which, idx_v):
    pltpu.sync_copy(ei_hbm.at[which, pl.ds(wid * EDGES_PER_W, EDGES_PER_W)],
                    idx_v.at[pl.ds(0, EDGES_PER_W)])

    @pl.when(wid >= TAIL_WID0)
    def _():
        base = (TAIL_ROW0 + wid - TAIL_WID0) * CHUNK
        pltpu.sync_copy(ei_hbm.at[which, pl.ds(base, CHUNK)],
                        idx_v.at[pl.ds(EDGES_PER_W, CHUNK)])


def _sc_degree_body(ei_hbm, ones_hbm, zeros_hbm, out_hbm,
                    dst_v, ones_v, acc_sh, sem):
    c = lax.axis_index("c")
    s = lax.axis_index("s")
    wid = c * NS + s
    _copy_tile_slice(s, zeros_hbm, acc_sh)
    plsc.subcore_barrier()
    _load_index_flat(wid, ei_hbm, 1, dst_v)
    pltpu.sync_copy(ones_hbm, ones_v)

    def idx(j):
        return dst_v.at[pl.ds(j * CHUNK, CHUNK)]

    def outer(i, _):
        j0 = i * NBUF
        for b in range(NBUF):
            pltpu.make_async_copy(ones_v, acc_sh.at[idx(j0 + b)],
                                  sem).start(add=True)
        for b in range(NBUF):
            pltpu.make_async_copy(ones_v, acc_sh.at[idx(j0 + b)],
                                  sem).wait()
        return 0

    lax.fori_loop(0, ROWS_PER_W // NBUF, outer, 0)

    @pl.when(wid >= TAIL_WID0)
    def _():
        pltpu.sync_copy(ones_v, acc_sh.at[idx(ROWS_PER_W)], add=True)

    plsc.subcore_barrier()
    pltpu.sync_copy(acc_sh.at[pl.ds(s * TILE_ROWS, TILE_ROWS), pl.ds(0, 8)],
                    out_hbm.at[c, pl.ds(s * TILE_ROWS, TILE_ROWS)])

    @pl.when(s == NS - 1)
    def _():
        pltpu.sync_copy(acc_sh.at[pl.ds(TAIL_BASE, TAIL), pl.ds(0, 8)],
                        out_hbm.at[c, pl.ds(TAIL_BASE, TAIL)])


def _sc_agg_body(d, ei_hbm, y_hbm, zeros_hbm, out_hbm,
                 src_v, dst_v, r0, r1, r2, r3, r4, r5, acc_sh,
                 g0, g1, g2, g3, g4, g5):
    rows = (r0, r1, r2, r3, r4, r5)
    gsems = (g0, g1, g2, g3, g4, g5)
    c = lax.axis_index("c")
    s = lax.axis_index("s")
    wid = c * NS + s
    _copy_tile_slice(s, zeros_hbm, acc_sh)
    plsc.subcore_barrier()
    _load_index_flat(wid, ei_hbm, 0, src_v)
    _load_index_flat(wid, ei_hbm, 1, dst_v)

    def idx(v, j):
        return v.at[pl.ds(j * CHUNK, CHUNK)]

    for b in range(NBUF):
        pltpu.async_copy(y_hbm.at[idx(src_v, b)], rows[b], gsems[b])

    def outer(i, _):
        j0 = i * NBUF
        for b in range(NBUF):
            j = j0 + b
            pltpu.make_async_copy(y_hbm.at[idx(src_v, j)], rows[b],
                                  gsems[b]).wait()
            pltpu.sync_copy(rows[b], acc_sh.at[idx(dst_v, j)], add=True)

            @pl.when(j + NBUF < ROWS_PER_W)
            def _():
                pltpu.async_copy(y_hbm.at[idx(src_v, j + NBUF)], rows[b],
                                 gsems[b])
        return 0

    lax.fori_loop(0, ROWS_PER_W // NBUF, outer, 0)

    @pl.when(wid >= TAIL_WID0)
    def _():
        pltpu.async_copy(y_hbm.at[idx(src_v, ROWS_PER_W)], rows[0],
                         gsems[0]).wait()
        pltpu.sync_copy(rows[0], acc_sh.at[idx(dst_v, ROWS_PER_W)],
                        add=True)

    plsc.subcore_barrier()
    _copy_tile_slice(s, acc_sh, out_hbm.at[c])


def _sc_degree(ei, ones, zeros):
    return pl.kernel(
        _sc_degree_body,
        out_type=jax.ShapeDtypeStruct((NC, N_NODES, 8), jnp.float32),
        mesh=_mesh,
        compiler_params=_sc_params,
        scratch_types=[
            pltpu.VMEM((EDGES_PER_W + CHUNK,), jnp.int32),
            pltpu.VMEM((CHUNK, DEG_W), jnp.float32),
            pltpu.VMEM_SHARED((N_NODES, DEG_W), jnp.float32),
            pltpu.SemaphoreType.DMA,
        ],
    )(ei, ones, zeros)


def _sc_agg(d, ei, y, zeros):
    return pl.kernel(
        functools.partial(_sc_agg_body, d),
        out_type=jax.ShapeDtypeStruct((NC, N_NODES, d), jnp.float32),
        mesh=_mesh,
        compiler_params=_sc_params,
        scratch_types=[
            pltpu.VMEM((EDGES_PER_W + CHUNK,), jnp.int32),
            pltpu.VMEM((EDGES_PER_W + CHUNK,), jnp.int32),
        ] + [pltpu.VMEM((CHUNK, d), jnp.float32)] * NBUF + [
            pltpu.VMEM_SHARED((N_NODES, d), jnp.float32),
        ] + [pltpu.SemaphoreType.DMA] * NBUF,
    )(ei, y, zeros)


def _tc1a_body(x_ref, w1_ref, xw_ref):
    xw_ref[...] = jnp.dot(x_ref[...], w1_ref[...],
                          preferred_element_type=jnp.float32)


def _tc1b_body(deg_ref, xw_ref, y_ref, dinv_ref):
    dacc = deg_ref[...]
    deg = dacc[0, :, :1] + dacc[1, :, :1] + 1.0
    dinv = lax.rsqrt(deg)
    y_ref[...] = xw_ref[...] * dinv
    dinv_ref[...] = dinv


def _tc2_body(acc_ref, y1_ref, dinv_ref, b1_ref, w2_ref, y2_ref):
    a = acc_ref[...]
    y1 = y1_ref[...]
    dinv = dinv_ref[...]
    h = jnp.maximum((a[0] + a[1] + y1) * dinv + b1_ref[...], 0.0)
    y2_ref[...] = jnp.dot(h, w2_ref[...],
                          preferred_element_type=jnp.float32) * dinv


def _tc3_body(acc_ref, y2_ref, dinv_ref, b2_ref, out_ref):
    a = acc_ref[...]
    o = jnp.maximum((a[0] + a[1] + y2_ref[...]) * dinv_ref[...] + b2_ref[...],
                    0.0)
    m = jnp.max(o, axis=1, keepdims=True)
    e = jnp.exp(o - m)
    out_ref[...] = e / jnp.sum(e, axis=1, keepdims=True)


def kernel(x, edge_index, W1, b1, W2, b2):
    ei = edge_index.astype(jnp.int32)
    ones = jnp.ones((CHUNK, DEG_W), jnp.float32)
    z1 = jnp.zeros((N_NODES, DEG_W), jnp.float32)
    z48 = jnp.zeros((N_NODES, W1.shape[1]), jnp.float32)
    z64 = jnp.zeros((N_NODES, W2.shape[1]), jnp.float32)

    xw = pl.pallas_call(
        _tc1a_body,
        out_shape=jax.ShapeDtypeStruct((N_NODES, W1.shape[1]), jnp.float32),
    )(x, W1)
    deg2 = _sc_degree(ei, ones, z1)

    y1, dinv = pl.pallas_call(
        _tc1b_body,
        out_shape=[
            jax.ShapeDtypeStruct((N_NODES, W1.shape[1]), jnp.float32),
            jax.ShapeDtypeStruct((N_NODES, 1), jnp.float32),
        ],
    )(deg2, xw)

    acc1 = _sc_agg(W1.shape[1], ei, y1, z48)

    y2 = pl.pallas_call(
        _tc2_body,
        out_shape=jax.ShapeDtypeStruct((N_NODES, W2.shape[1]), jnp.float32),
    )(acc1, y1, dinv, b1, W2)

    acc2 = _sc_agg(W2.shape[1], ei, y2, z64)

    out = pl.pallas_call(
        _tc3_body,
        out_shape=jax.ShapeDtypeStruct((N_NODES, W2.shape[1]), jnp.float32),
    )(acc2, y2, dinv, b2)
    return out


# speedup vs baseline: 1.0079x; 1.0079x over previous
import functools

import jax
import jax.numpy as jnp
from jax import lax
from jax.experimental import pallas as pl
from jax.experimental.pallas import tpu as pltpu
from jax.experimental.pallas import tpu_sc as plsc

N_NODES = 10000
N_EDGES = 320000
NC = 2
NS = 16
NW = NC * NS
CHUNK = 128
N_ROWS = N_EDGES // CHUNK
ROWS_PER_W = N_ROWS // NW
TAIL_ROW0 = NW * ROWS_PER_W
TAIL_WID0 = NW - (N_ROWS - TAIL_ROW0)
NBUF = 6
DEG_W = 16
TILE_ROWS = 624
TAIL_BASE = NS * TILE_ROWS
TAIL = N_NODES - TAIL_BASE

_mesh = plsc.VectorSubcoreMesh(core_axis_name="c", subcore_axis_name="s")
_sc_params = pltpu.CompilerParams(use_tc_tiling_on_sc=False)


def _copy_tile_slice(s, src, dst):
    pltpu.sync_copy(src.at[pl.ds(s * TILE_ROWS, TILE_ROWS)],
                    dst.at[pl.ds(s * TILE_ROWS, TILE_ROWS)])

    @pl.when(s == NS - 1)
    def _():
        pltpu.sync_copy(src.at[pl.ds(TAIL_BASE, TAIL)],
                        dst.at[pl.ds(TAIL_BASE, TAIL)])


EDGES_PER_W = ROWS_PER_W * CHUNK


def _load_index_flat(wid, ei_hbm, which, idx_v):
    pltpu.sync_copy(ei_hbm.at[which, pl.ds(wid * EDGES_PER_W, EDGES_PER_W)],
                    idx_v.at[pl.ds(0, EDGES_PER_W)])

    @pl.when(wid >= TAIL_WID0)
    def _():
        base = (TAIL_ROW0 + wid - TAIL_WID0) * CHUNK
        pltpu.sync_copy(ei_hbm.at[which, pl.ds(base, CHUNK)],
                        idx_v.at[pl.ds(EDGES_PER_W, CHUNK)])


def _sc_degree_body(ei_hbm, ones_hbm, zeros_hbm, out_hbm,
                    dst_v, ones_v, acc_sh, sem):
    c = lax.axis_index("c")
    s = lax.axis_index("s")
    wid = c * NS + s
    _copy_tile_slice(s, zeros_hbm, acc_sh)
    plsc.subcore_barrier()
    _load_index_flat(wid, ei_hbm, 1, dst_v)
    pltpu.sync_copy(ones_hbm, ones_v)

    def idx(j):
        return dst_v.at[pl.ds(j * CHUNK, CHUNK)]

    def outer(i, _):
        j0 = i * NBUF
        for b in range(NBUF):
            pltpu.make_async_copy(ones_v, acc_sh.at[idx(j0 + b)],
                                  sem).start(add=True)
        for b in range(NBUF):
            pltpu.make_async_copy(ones_v, acc_sh.at[idx(j0 + b)],
                                  sem).wait()
        return 0

    lax.fori_loop(0, ROWS_PER_W // NBUF, outer, 0)

    @pl.when(wid >= TAIL_WID0)
    def _():
        pltpu.sync_copy(ones_v, acc_sh.at[idx(ROWS_PER_W)], add=True)

    plsc.subcore_barrier()
    pltpu.sync_copy(acc_sh.at[pl.ds(s * TILE_ROWS, TILE_ROWS), pl.ds(0, 8)],
                    out_hbm.at[c, pl.ds(s * TILE_ROWS, TILE_ROWS)])

    @pl.when(s == NS - 1)
    def _():
        pltpu.sync_copy(acc_sh.at[pl.ds(TAIL_BASE, TAIL), pl.ds(0, 8)],
                        out_hbm.at[c, pl.ds(TAIL_BASE, TAIL)])


def _sc_agg_body(d, ei_hbm, y_hbm, zeros_hbm, out_hbm,
                 src_v, dst_v, r0, r1, r2, r3, r4, r5, acc_sh,
                 g0, g1, g2, g3, g4, g5):
    rows = (r0, r1, r2, r3, r4, r5)
    gsems = (g0, g1, g2, g3, g4, g5)
    c = lax.axis_index("c")
    s = lax.axis_index("s")
    wid = c * NS + s
    _copy_tile_slice(s, zeros_hbm, acc_sh)
    plsc.subcore_barrier()
    _load_index_flat(wid, ei_hbm, 0, src_v)
    _load_index_flat(wid, ei_hbm, 1, dst_v)

    def idx(v, j):
        return v.at[pl.ds(j * CHUNK, CHUNK)]

    for b in range(NBUF):
        pltpu.async_copy(y_hbm.at[idx(src_v, b)], rows[b], gsems[b])

    def outer(i, _):
        j0 = i * NBUF
        for b in range(NBUF):
            j = j0 + b
            pltpu.make_async_copy(y_hbm.at[idx(src_v, j)], rows[b],
                                  gsems[b]).wait()
            pltpu.sync_copy(rows[b], acc_sh.at[idx(dst_v, j)], add=True)

            @pl.when(j + NBUF < ROWS_PER_W)
            def _():
                pltpu.async_copy(y_hbm.at[idx(src_v, j + NBUF)], rows[b],
                                 gsems[b])
        return 0

    lax.fori_loop(0, ROWS_PER_W // NBUF, outer, 0)

    @pl.when(wid >= TAIL_WID0)
    def _():
        pltpu.async_copy(y_hbm.at[idx(src_v, ROWS_PER_W)], rows[0],
                         gsems[0]).wait()
        pltpu.sync_copy(rows[0], acc_sh.at[idx(dst_v, ROWS_PER_W)],
                        add=True)

    plsc.subcore_barrier()
    _copy_tile_slice(s, acc_sh, out_hbm.at[c])


def _sc_degree(ei, ones, zeros):
    return pl.kernel(
        _sc_degree_body,
        out_type=jax.ShapeDtypeStruct((NC, N_NODES, 8), jnp.float32),
        mesh=_mesh,
        compiler_params=_sc_params,
        scratch_types=[
            pltpu.VMEM((EDGES_PER_W + CHUNK,), jnp.int32),
            pltpu.VMEM((CHUNK, DEG_W), jnp.float32),
            pltpu.VMEM_SHARED((N_NODES, DEG_W), jnp.float32),
            pltpu.SemaphoreType.DMA,
        ],
    )(ei, ones, zeros)


def _sc_agg(d, ei, y, zeros):
    return pl.kernel(
        functools.partial(_sc_agg_body, d),
        out_type=jax.ShapeDtypeStruct((NC, N_NODES, d), jnp.float32),
        mesh=_mesh,
        compiler_params=_sc_params,
        scratch_types=[
            pltpu.VMEM((EDGES_PER_W + CHUNK,), jnp.int32),
            pltpu.VMEM((EDGES_PER_W + CHUNK,), jnp.int32),
        ] + [pltpu.VMEM((CHUNK, d), jnp.float32)] * NBUF + [
            pltpu.VMEM_SHARED((N_NODES, d), jnp.float32),
        ] + [pltpu.SemaphoreType.DMA] * NBUF,
    )(ei, y, zeros)


def _tc1_body(deg_ref, x_ref, w1_ref, y_ref, dinv_ref):
    dacc = deg_ref[...]
    deg = dacc[0, :, :1] + dacc[1, :, :1] + 1.0
    dinv = lax.rsqrt(deg)
    xw = jnp.dot(x_ref[...], w1_ref[...], preferred_element_type=jnp.float32)
    y_ref[...] = xw * dinv
    dinv_ref[...] = dinv


def _tc2_body(acc_ref, y1_ref, dinv_ref, b1_ref, w2_ref, y2_ref):
    a = acc_ref[...]
    y1 = y1_ref[...]
    dinv = dinv_ref[...]
    h = jnp.maximum((a[0] + a[1] + y1) * dinv + b1_ref[...], 0.0)
    y2_ref[...] = jnp.dot(h, w2_ref[...],
                          preferred_element_type=jnp.float32) * dinv


def _tc3_body(acc_ref, y2_ref, dinv_ref, b2_ref, out_ref):
    a = acc_ref[...]
    o = jnp.maximum((a[0] + a[1] + y2_ref[...]) * dinv_ref[...] + b2_ref[...],
                    0.0)
    m = jnp.max(o, axis=1, keepdims=True)
    e = jnp.exp(o - m)
    out_ref[...] = e / jnp.sum(e, axis=1, keepdims=True)


def kernel(x, edge_index, W1, b1, W2, b2):
    ei = edge_index.astype(jnp.int32)
    ones = jnp.ones((CHUNK, DEG_W), jnp.float32)
    z1 = jnp.zeros((N_NODES, DEG_W), jnp.float32)
    z48 = jnp.zeros((N_NODES, W1.shape[1]), jnp.float32)
    z64 = jnp.zeros((N_NODES, W2.shape[1]), jnp.float32)

    deg2 = _sc_degree(ei, ones, z1)

    y1, dinv = pl.pallas_call(
        _tc1_body,
        out_shape=[
            jax.ShapeDtypeStruct((N_NODES, W1.shape[1]), jnp.float32),
            jax.ShapeDtypeStruct((N_NODES, 1), jnp.float32),
        ],
    )(deg2, x, W1)

    acc1 = _sc_agg(W1.shape[1], ei, y1, z48)

    y2 = pl.pallas_call(
        _tc2_body,
        out_shape=jax.ShapeDtypeStruct((N_NODES, W2.shape[1]), jnp.float32),
    )(acc1, y1, dinv, b1, W2)

    acc2 = _sc_agg(W2.shape[1], ei, y2, z64)

    out = pl.pallas_call(
        _tc3_body,
        out_shape=jax.ShapeDtypeStruct((N_NODES, W2.shape[1]), jnp.float32),
    )(acc2, y2, dinv, b2)
    return out


# speedup vs baseline: 1.0372x; 1.0291x over previous
import functools

import jax
import jax.numpy as jnp
from jax import lax
from jax.experimental import pallas as pl
from jax.experimental.pallas import tpu as pltpu
from jax.experimental.pallas import tpu_sc as plsc

N_NODES = 10000
N_EDGES = 320000
NC = 2
NS = 16
NW = NC * NS
CHUNK = 128
N_ROWS = N_EDGES // CHUNK
ROWS_PER_W = N_ROWS // NW
TAIL_ROW0 = NW * ROWS_PER_W
TAIL_WID0 = NW - (N_ROWS - TAIL_ROW0)
NBUF = 6
DEG_W = 16
TILE_ROWS = 624
TAIL_BASE = NS * TILE_ROWS
TAIL = N_NODES - TAIL_BASE

_mesh = plsc.VectorSubcoreMesh(core_axis_name="c", subcore_axis_name="s")
_sc_params = pltpu.CompilerParams(use_tc_tiling_on_sc=False)


def _copy_tile_slice(s, src, dst):
    pltpu.sync_copy(src.at[pl.ds(s * TILE_ROWS, TILE_ROWS)],
                    dst.at[pl.ds(s * TILE_ROWS, TILE_ROWS)])

    @pl.when(s == NS - 1)
    def _():
        pltpu.sync_copy(src.at[pl.ds(TAIL_BASE, TAIL)],
                        dst.at[pl.ds(TAIL_BASE, TAIL)])


EDGES_PER_W = ROWS_PER_W * CHUNK


def _load_index_flat(wid, ei_hbm, which, idx_v):
    pltpu.sync_copy(ei_hbm.at[which, pl.ds(wid * EDGES_PER_W, EDGES_PER_W)],
                    idx_v.at[pl.ds(0, EDGES_PER_W)])

    @pl.when(wid >= TAIL_WID0)
    def _():
        base = (TAIL_ROW0 + wid - TAIL_WID0) * CHUNK
        pltpu.sync_copy(ei_hbm.at[which, pl.ds(base, CHUNK)],
                        idx_v.at[pl.ds(EDGES_PER_W, CHUNK)])


def _sc_degree_body(ei_hbm, ones_hbm, zeros_hbm, out_hbm,
                    dst_v, ones_v, acc_sh, sem):
    c = lax.axis_index("c")
    s = lax.axis_index("s")
    wid = c * NS + s
    _copy_tile_slice(s, zeros_hbm, acc_sh)
    plsc.subcore_barrier()
    _load_index_flat(wid, ei_hbm, 1, dst_v)
    pltpu.sync_copy(ones_hbm, ones_v)

    def idx(j):
        return dst_v.at[pl.ds(j * CHUNK, CHUNK)]

    def outer(i, _):
        j0 = i * NBUF
        for b in range(NBUF):
            pltpu.make_async_copy(ones_v, acc_sh.at[idx(j0 + b)],
                                  sem).start(add=True)
        for b in range(NBUF):
            pltpu.make_async_copy(ones_v, acc_sh.at[idx(j0 + b)],
                                  sem).wait()
        return 0

    lax.fori_loop(0, ROWS_PER_W // NBUF, outer, 0)

    @pl.when(wid >= TAIL_WID0)
    def _():
        pltpu.sync_copy(ones_v, acc_sh.at[idx(ROWS_PER_W)], add=True)

    plsc.subcore_barrier()
    _copy_tile_slice(s, acc_sh, out_hbm.at[c])


def _sc_agg_body(d, ei_hbm, y_hbm, zeros_hbm, out_hbm,
                 src_v, dst_v, r0, r1, r2, r3, r4, r5, acc_sh,
                 g0, g1, g2, g3, g4, g5):
    rows = (r0, r1, r2, r3, r4, r5)
    gsems = (g0, g1, g2, g3, g4, g5)
    c = lax.axis_index("c")
    s = lax.axis_index("s")
    wid = c * NS + s
    _copy_tile_slice(s, zeros_hbm, acc_sh)
    plsc.subcore_barrier()
    _load_index_flat(wid, ei_hbm, 0, src_v)
    _load_index_flat(wid, ei_hbm, 1, dst_v)

    def idx(v, j):
        return v.at[pl.ds(j * CHUNK, CHUNK)]

    for b in range(NBUF):
        pltpu.async_copy(y_hbm.at[idx(src_v, b)], rows[b], gsems[b])

    def outer(i, _):
        j0 = i * NBUF
        for b in range(NBUF):
            j = j0 + b
            pltpu.make_async_copy(y_hbm.at[idx(src_v, j)], rows[b],
                                  gsems[b]).wait()
            pltpu.sync_copy(rows[b], acc_sh.at[idx(dst_v, j)], add=True)

            @pl.when(j + NBUF < ROWS_PER_W)
            def _():
                pltpu.async_copy(y_hbm.at[idx(src_v, j + NBUF)], rows[b],
                                 gsems[b])
        return 0

    lax.fori_loop(0, ROWS_PER_W // NBUF, outer, 0)

    @pl.when(wid >= TAIL_WID0)
    def _():
        pltpu.async_copy(y_hbm.at[idx(src_v, ROWS_PER_W)], rows[0],
                         gsems[0]).wait()
        pltpu.sync_copy(rows[0], acc_sh.at[idx(dst_v, ROWS_PER_W)],
                        add=True)

    plsc.subcore_barrier()
    _copy_tile_slice(s, acc_sh, out_hbm.at[c])


def _sc_degree(ei, ones, zeros):
    return pl.kernel(
        _sc_degree_body,
        out_type=jax.ShapeDtypeStruct((NC, N_NODES, DEG_W), jnp.float32),
        mesh=_mesh,
        compiler_params=_sc_params,
        scratch_types=[
            pltpu.VMEM((EDGES_PER_W + CHUNK,), jnp.int32),
            pltpu.VMEM((CHUNK, DEG_W), jnp.float32),
            pltpu.VMEM_SHARED((N_NODES, DEG_W), jnp.float32),
            pltpu.SemaphoreType.DMA,
        ],
    )(ei, ones, zeros)


def _sc_agg(d, ei, y, zeros):
    return pl.kernel(
        functools.partial(_sc_agg_body, d),
        out_type=jax.ShapeDtypeStruct((NC, N_NODES, d), jnp.float32),
        mesh=_mesh,
        compiler_params=_sc_params,
        scratch_types=[
            pltpu.VMEM((EDGES_PER_W + CHUNK,), jnp.int32),
            pltpu.VMEM((EDGES_PER_W + CHUNK,), jnp.int32),
        ] + [pltpu.VMEM((CHUNK, d), jnp.float32)] * NBUF + [
            pltpu.VMEM_SHARED((N_NODES, d), jnp.float32),
        ] + [pltpu.SemaphoreType.DMA] * NBUF,
    )(ei, y, zeros)


def _tc1_body(deg_ref, x_ref, w1_ref, y_ref, dinv_ref):
    dacc = deg_ref[...]
    deg = dacc[0, :, :1] + dacc[1, :, :1] + 1.0
    dinv = lax.rsqrt(deg)
    xw = jnp.dot(x_ref[...], w1_ref[...], preferred_element_type=jnp.float32)
    y_ref[...] = xw * dinv
    dinv_ref[...] = dinv


def _tc2_body(acc_ref, y1_ref, dinv_ref, b1_ref, w2_ref, y2_ref):
    a = acc_ref[...]
    y1 = y1_ref[...]
    dinv = dinv_ref[...]
    h = jnp.maximum((a[0] + a[1] + y1) * dinv + b1_ref[...], 0.0)
    y2_ref[...] = jnp.dot(h, w2_ref[...],
                          preferred_element_type=jnp.float32) * dinv


def _tc3_body(acc_ref, y2_ref, dinv_ref, b2_ref, out_ref):
    a = acc_ref[...]
    o = jnp.maximum((a[0] + a[1] + y2_ref[...]) * dinv_ref[...] + b2_ref[...],
                    0.0)
    m = jnp.max(o, axis=1, keepdims=True)
    e = jnp.exp(o - m)
    out_ref[...] = e / jnp.sum(e, axis=1, keepdims=True)


def kernel(x, edge_index, W1, b1, W2, b2):
    ei = edge_index.astype(jnp.int32)
    ones = jnp.ones((CHUNK, DEG_W), jnp.float32)
    z1 = jnp.zeros((N_NODES, DEG_W), jnp.float32)
    z48 = jnp.zeros((N_NODES, W1.shape[1]), jnp.float32)
    z64 = jnp.zeros((N_NODES, W2.shape[1]), jnp.float32)

    deg2 = _sc_degree(ei, ones, z1)

    y1, dinv = pl.pallas_call(
        _tc1_body,
        out_shape=[
            jax.ShapeDtypeStruct((N_NODES, W1.shape[1]), jnp.float32),
            jax.ShapeDtypeStruct((N_NODES, 1), jnp.float32),
        ],
    )(deg2, x, W1)

    acc1 = _sc_agg(W1.shape[1], ei, y1, z48)

    y2 = pl.pallas_call(
        _tc2_body,
        out_shape=jax.ShapeDtypeStruct((N_NODES, W2.shape[1]), jnp.float32),
    )(acc1, y1, dinv, b1, W2)

    acc2 = _sc_agg(W2.shape[1], ei, y2, z64)

    out = pl.pallas_call(
        _tc3_body,
        out_shape=jax.ShapeDtypeStruct((N_NODES, W2.shape[1]), jnp.float32),
    )(acc2, y2, dinv, b2)
    return out
